# baseline (speedup 1.0000x reference)
import functools

import jax
import jax.numpy as jnp
from jax import lax
from jax.experimental import pallas as pl
from jax.experimental.pallas import tpu as pltpu
from jax.experimental.pallas import tpu_sc as plsc

_L = 16
_NW = 32
_IDX_CHUNK = 128


def _sqrt16(x):
    xs = jnp.maximum(x, jnp.float32(1e-30))
    i = lax.bitcast_convert_type(xs, jnp.int32)
    i = jnp.int32(0x5F3759DF) - lax.shift_right_arithmetic(i, jnp.int32(1))
    y = lax.bitcast_convert_type(i, jnp.float32)
    half = jnp.float32(0.5)
    three_half = jnp.float32(1.5)
    for _ in range(3):
        y = y * (three_half - half * xs * y * y)
    return x * y


def _make_sc_call(batch, dim):
    bpw = batch // _NW
    nch = bpw // _IDX_CHUNK
    ngr = bpw // _L
    rows_per_w = bpw // _IDX_CHUNK

    mesh = plsc.VectorSubcoreMesh(core_axis_name="c", subcore_axis_name="s")

    @functools.partial(
        pl.kernel,
        out_type=jax.ShapeDtypeStruct((batch,), jnp.float32),
        mesh=mesh,
        compiler_params=pltpu.CompilerParams(
            needs_layout_passes=False, use_tc_tiling_on_sc=False),
        scratch_types=[
            pltpu.VMEM((rows_per_w, _IDX_CHUNK), jnp.int32),
            pltpu.VMEM((rows_per_w, _IDX_CHUNK), jnp.int32),
            pltpu.VMEM((rows_per_w, _IDX_CHUNK), jnp.int32),
            pltpu.VMEM((bpw, dim), jnp.float32),
            pltpu.VMEM((bpw, dim), jnp.float32),
            pltpu.VMEM((bpw, dim), jnp.float32),
            pltpu.VMEM((bpw,), jnp.float32),
            pltpu.VMEM((dim, _L), jnp.float32),
            pltpu.VMEM((bpw,), jnp.float32),
            pltpu.SemaphoreType.DMA,
        ],
    )
    def sc_call(uid_hbm, lit_hbm, pit_hbm, uemb_hbm, iemb_hbm, gt_hbm,
                bias_hbm, out_hbm, idx_u, idx_l, idx_p, rows_u, rows_l,
                rows_p, bias_v, gt_v, out_v, sem):
        wid = lax.axis_index("s") * 2 + lax.axis_index("c")
        base_row = wid * rows_per_w

        pltpu.sync_copy(uid_hbm.at[pl.ds(base_row, rows_per_w)], idx_u)
        pltpu.sync_copy(lit_hbm.at[pl.ds(base_row, rows_per_w)], idx_l)
        pltpu.sync_copy(pit_hbm.at[pl.ds(base_row, rows_per_w)], idx_p)
        pltpu.sync_copy(gt_hbm, gt_v)

        copies = []
        for j in range(nch):
            dst = pl.ds(j * _IDX_CHUNK, _IDX_CHUNK)
            copies.append(pltpu.async_copy(
                uemb_hbm.at[idx_u.at[j]], rows_u.at[dst], sem))
            copies.append(pltpu.async_copy(
                iemb_hbm.at[idx_l.at[j]], rows_l.at[dst], sem))
            copies.append(pltpu.async_copy(
                iemb_hbm.at[idx_p.at[j]], rows_p.at[dst], sem))
            copies.append(pltpu.async_copy(
                bias_hbm.at[idx_p.at[j]], bias_v.at[dst], sem))
        for c in copies:
            c.wait()

        lane = lax.iota(jnp.int32, _L)

        def group_body(g, carry):
            rows = g * _L + lane
            acc = jnp.zeros((_L,), jnp.float32)
            for k in range(dim):
                col = jnp.full((_L,), k, jnp.int32)
                u = plsc.load_gather(rows_u, [rows, col])
                li = plsc.load_gather(rows_l, [rows, col])
                p = plsc.load_gather(rows_p, [rows, col])
                d = (u - p) + li + gt_v[k]
                acc = acc + d * d
            b = bias_v[pl.ds(g * _L, _L)]
            out_v[pl.ds(g * _L, _L)] = b - _sqrt16(acc)
            return carry

        lax.fori_loop(0, ngr, group_body, jnp.int32(0))

        pltpu.sync_copy(out_v, out_hbm.at[pl.ds(wid * bpw, bpw)])

    return sc_call


def kernel(user_ids, last_items, pre_items, user_emb, item_emb,
           global_transition, item_biases):
    batch = user_ids.shape[0]
    dim = user_emb.shape[1]
    uid2 = user_ids.astype(jnp.int32).reshape(-1, _IDX_CHUNK)
    lit2 = last_items.astype(jnp.int32).reshape(-1, _IDX_CHUNK)
    pit2 = pre_items.astype(jnp.int32).reshape(-1, _IDX_CHUNK)
    gt_cols = jnp.broadcast_to(
        global_transition.astype(jnp.float32).reshape(dim, 1), (dim, _L))
    bias1 = item_biases.astype(jnp.float32).reshape(-1)
    sc_call = _make_sc_call(batch, dim)
    return sc_call(uid2, lit2, pit2, user_emb, item_emb, gt_cols, bias1)
